# Initial kernel scaffold; baseline (speedup 1.0000x reference)
import jax
import jax.numpy as jnp
from jax.experimental import pallas as pl


def kernel(data_view, label, inits_view):
    raise NotImplementedError("write your pallas kernel here")


# speedup vs baseline: 2.2492x; 2.2492x over previous
import functools

import jax
import jax.numpy as jnp
from jax import lax
from jax.experimental import pallas as pl
from jax.experimental.pallas import tpu as pltpu
from jax.experimental.pallas import tpu_sc as plsc

B, S, D = 16, 2048, 1024
R = B * S
NC, NS = 2, 16
NW = NC * NS
RPW = R // NW
C = 32
NCHUNK = RPW // C


_MESH = plsc.VectorSubcoreMesh(core_axis_name="c", subcore_axis_name="s")


@functools.partial(
    pl.kernel,
    mesh=_MESH,
    out_type=[
        jax.ShapeDtypeStruct((R, D), jnp.float32),
        jax.ShapeDtypeStruct((R, D), jnp.float32),
        jax.ShapeDtypeStruct((R,), jnp.float32),
    ],
    scratch_types=[
        pltpu.VMEM((NCHUNK, C), jnp.int32),
        pltpu.VMEM((C, D), jnp.float32),
        pltpu.VMEM((R,), jnp.float32),
        pltpu.VMEM((RPW,), jnp.float32),
        pltpu.SemaphoreType.DMA,
    ],
    compiler_params=pltpu.CompilerParams(needs_layout_passes=False),
)
def _sc_shuffle(data_hbm, inits_hbm, label_hbm, idx_hbm,
                dout_hbm, iout_hbm, lout_hbm,
                idx_v, buf, lab_v, lout_v, sem):
    wid = lax.axis_index("s") * NC + lax.axis_index("c")
    base = wid * RPW

    pltpu.sync_copy(idx_hbm.at[wid], idx_v)

    def gather_tensor(src_hbm, dst_hbm):
        def step(j, carry):
            pltpu.async_copy(src_hbm.at[idx_v.at[j]], buf, sem).wait()
            pltpu.sync_copy(buf, dst_hbm.at[pl.ds(base + j * C, C)])
            return carry
        lax.fori_loop(0, NCHUNK, step, 0)

    gather_tensor(data_hbm, dout_hbm)
    gather_tensor(inits_hbm, iout_hbm)

    pltpu.sync_copy(label_hbm, lab_v)

    def lstep(j, carry):
        for h in range(C // 16):
            ivec = idx_v[j, pl.ds(h * 16, 16)]
            lout_v[pl.ds(j * C + h * 16, 16)] = plsc.load_gather(lab_v, [ivec])
        return carry
    lax.fori_loop(0, NCHUNK, lstep, 0)
    pltpu.sync_copy(lout_v, lout_hbm.at[pl.ds(base, RPW)])


def kernel(data_view, label, inits_view):
    perms = jax.random.permutation(jax.random.key(42), S)
    row_idx = (jnp.arange(B, dtype=jnp.int32)[:, None] * S
               + perms[None, :].astype(jnp.int32))
    idx = row_idx.reshape(NW, NCHUNK, C)
    dout, iout, lout = _sc_shuffle(
        data_view.reshape(R, D), inits_view.reshape(R, D),
        label.reshape(R), idx)
    return (dout.reshape(B, S, D), lout, iout.reshape(B, S, D), perms)
